# baseline (speedup 1.0000x reference)
import functools
import jax
import jax.numpy as jnp
from jax import lax
from jax.experimental import pallas as pl
from jax.experimental.pallas import tpu as pltpu
from jax.experimental.pallas import tpu_sc as plsc

X_MIN = -3.0
X_MAX = 3.0
KN = 64
IN = 64
ON = 256
H = (X_MAX - X_MIN) / (KN - 1)

NW = 32
BSC = 256


def _pchip(y):
    delta = (y[1:, :] - y[:-1, :]) * (1.0 / H)
    d0 = (3.0 * delta[0:1, :] - delta[1:2, :]) * 0.5
    dN = (3.0 * delta[KN - 2:KN - 1, :] - delta[KN - 3:KN - 2, :]) * 0.5

    def fix_end(d_end, delta0, delta1):
        d_end = jnp.where(d_end * delta0 <= 0.0, 0.0, d_end)
        bad = (delta0 * delta1 < 0.0) & (jnp.abs(d_end) > 3.0 * jnp.abs(delta0))
        return jnp.where(bad, 3.0 * delta0, d_end)

    d0 = fix_end(d0, delta[0:1, :], delta[1:2, :])
    dN = fix_end(dN, delta[KN - 2:KN - 1, :], delta[KN - 3:KN - 2, :])
    dp = delta[:-1, :]
    dn = delta[1:, :]
    same = dp * dn > 0.0
    dmid = jnp.where(same, 2.0 * dp * dn / (dp + dn + 1e-12), 0.0)
    return H * jnp.concatenate([d0, dmid, dN], axis=0)


def _prep_t1_body(ct_ref, t1_ref):
    y = ct_ref[...]
    hd = _pchip(y)
    t1_ref[:, 0:ON] = y
    t1_ref[:, ON:2 * ON] = hd


def _prep_gt_body(ct_ref, gt_ref):
    y = ct_ref[...]
    hd = _pchip(y)
    gt_ref[:, 0:KN] = jnp.transpose(y)
    gt_ref[:, KN:2 * KN] = jnp.transpose(hd)


def _weights_body(x_ref, ridx_ref, w_ref):
    x = x_ref[...]
    t = (x - X_MIN) * (1.0 / H)
    idx = jnp.clip(jnp.floor(t), 0.0, float(KN - 2))
    u = t - idx
    u2 = u * u
    u3 = u2 * u
    h00 = 2.0 * u3 - 3.0 * u2 + 1.0
    h10 = u3 - 2.0 * u2 + u
    h01 = 3.0 * u2 - 2.0 * u3
    h11 = u3 - u2
    left = t < 0.0
    right = t > float(KN - 1)
    wy0 = jnp.where(left, 1.0, jnp.where(right, 0.0, h00))
    wd0 = jnp.where(left, t, jnp.where(right, 0.0, h10))
    wy1 = jnp.where(left, 0.0, jnp.where(right, 1.0, h01))
    wd1 = jnp.where(left, 0.0, jnp.where(right, u - 1.0, h11))
    feat = jax.lax.broadcasted_iota(jnp.int32, x.shape, 1)
    rid = feat * KN + idx.astype(jnp.int32)
    ridx_ref[:, 0:IN] = rid
    ridx_ref[:, IN:2 * IN] = rid + 1
    w_ref[:, 0:IN] = wy0
    w_ref[:, IN:2 * IN] = wd0
    w_ref[:, 2 * IN:3 * IN] = wy1
    w_ref[:, 3 * IN:4 * IN] = wd1


def _sc_bag_body(t1_hbm, ridx_hbm, w_hbm, bias_hbm, out_hbm,
                 idx_v, w_v, rows_v, bias_v, out_v, sem_g):
    wid = lax.axis_index("s") * 2 + lax.axis_index("c")
    nb = BSC // NW
    base = wid * nb
    pltpu.sync_copy(bias_hbm, bias_v)

    def body_b(bb, _):
        b = base + bb
        pltpu.sync_copy(ridx_hbm.at[b], idx_v)
        pltpu.sync_copy(w_hbm.at[b], w_v.at[pl.ds(0, 4 * IN)])
        cp0 = pltpu.async_copy(
            t1_hbm.at[idx_v.at[pl.ds(0, IN)]],
            rows_v.at[pl.ds(0, IN)], sem_g)
        cp1 = pltpu.async_copy(
            t1_hbm.at[idx_v.at[pl.ds(IN, IN)]],
            rows_v.at[pl.ds(IN, IN)], sem_g)

        acc0 = tuple(bias_v[pl.ds(c * 16, 16)] for c in range(16))

        def make_body(row_off, wy_off, wd_off):
            def body_i(i, acc):
                wy = jnp.full((16,), w_v[pl.ds(wy_off + i, 16)][0],
                              jnp.float32)
                wd = jnp.full((16,), w_v[pl.ds(wd_off + i, 16)][0],
                              jnp.float32)
                new = []
                for c in range(16):
                    a = acc[c]
                    a = a + wy * rows_v[row_off + i, pl.ds(c * 16, 16)]
                    a = a + wd * rows_v[row_off + i, pl.ds(ON + c * 16, 16)]
                    new.append(a)
                return tuple(new)
            return body_i

        cp0.wait()
        acc = lax.fori_loop(0, IN, make_body(0, 0, IN), acc0)
        cp1.wait()
        acc = lax.fori_loop(0, IN, make_body(IN, 2 * IN, 3 * IN), acc)
        for c in range(16):
            out_v[pl.ds(c * 16, 16)] = acc[c]
        pltpu.sync_copy(out_v, out_hbm.at[b])
        return 0

    lax.fori_loop(0, nb, body_b, 0)


def _onehot_t_body(xt_ref, gt_ref, b_ref, ot_ref):
    x = xt_ref[...]
    t = (x - X_MIN) * (1.0 / H)
    idx = jnp.clip(jnp.floor(t), 0.0, float(KN - 2))
    u = t - idx
    u2 = u * u
    u3 = u2 * u
    h00 = 2.0 * u3 - 3.0 * u2 + 1.0
    h10 = u3 - 2.0 * u2 + u
    h01 = 3.0 * u2 - 2.0 * u3
    h11 = u3 - u2
    left = t < 0.0
    right = t > float(KN - 1)
    wy0 = jnp.where(left, 1.0, jnp.where(right, 0.0, h00))
    wd0 = jnp.where(left, t, jnp.where(right, 0.0, h10))
    wy1 = jnp.where(left, 0.0, jnp.where(right, 1.0, h01))
    wd1 = jnp.where(left, 0.0, jnp.where(right, u - 1.0, h11))

    btc = x.shape[1]
    idx3 = idx.astype(jnp.int32)[:, None, :]
    kk = jax.lax.broadcasted_iota(jnp.int32, (1, 2 * KN, 1), 1)
    kmod = jnp.where(kk < KN, kk, kk - KN)
    isy = kk < KN
    wlo = jnp.where(isy, wy0[:, None, :], wd0[:, None, :])
    whi = jnp.where(isy, wy1[:, None, :], wd1[:, None, :])
    s = jnp.where(kmod == idx3, wlo,
                  jnp.where(kmod == idx3 + 1, whi, 0.0))
    s2 = s.reshape(IN * 2 * KN, btc)
    acc = jax.lax.dot_general(
        gt_ref[...], s2, (((1,), (0,)), ((), ())),
        preferred_element_type=jnp.float32,
        precision=jax.lax.Precision.DEFAULT)
    ot_ref[...] = jnp.transpose(acc) + b_ref[...]


@jax.jit
def _run(x, coeffs, bias):
    ct = jnp.transpose(coeffs, (1, 2, 0)).reshape(IN * KN, ON)
    B = x.shape[0]
    BT = min(512, BSC)
    ridx, w4 = pl.pallas_call(
        _weights_body,
        grid=(BSC // BT,),
        in_specs=[pl.BlockSpec((BT, IN), lambda i: (i, 0))],
        out_specs=[pl.BlockSpec((BT, 2 * IN), lambda i: (i, 0)),
                   pl.BlockSpec((BT, 4 * IN), lambda i: (i, 0))],
        out_shape=[jax.ShapeDtypeStruct((BSC, 2 * IN), jnp.int32),
                   jax.ShapeDtypeStruct((BSC, 4 * IN), jnp.float32)],
    )(x[:BSC])

    t1 = pl.pallas_call(
        _prep_t1_body,
        grid=(IN,),
        in_specs=[pl.BlockSpec((KN, ON), lambda i: (i, 0))],
        out_specs=pl.BlockSpec((KN, 2 * ON), lambda i: (i, 0)),
        out_shape=jax.ShapeDtypeStruct((IN * KN, 2 * ON), jnp.float32),
    )(ct)

    mesh = plsc.VectorSubcoreMesh(core_axis_name="c", subcore_axis_name="s")
    sc_bag = functools.partial(
        pl.kernel,
        out_type=jax.ShapeDtypeStruct((BSC, ON), jnp.float32),
        mesh=mesh,
        scratch_types=[
            pltpu.VMEM((2 * IN,), jnp.int32),
            pltpu.VMEM((4 * IN + 16,), jnp.float32),
            pltpu.VMEM((2 * IN, 2 * ON), jnp.float32),
            pltpu.VMEM((ON,), jnp.float32),
            pltpu.VMEM((ON,), jnp.float32),
            pltpu.SemaphoreType.DMA,
        ],
    )(_sc_bag_body)
    out_sc = sc_bag(t1, ridx, w4, bias)

    gt = pl.pallas_call(
        _prep_gt_body,
        grid=(IN,),
        in_specs=[pl.BlockSpec((KN, ON), lambda i: (i, 0))],
        out_specs=pl.BlockSpec((ON, 2 * KN), lambda i: (0, i)),
        out_shape=jax.ShapeDtypeStruct((ON, IN * 2 * KN), jnp.float32),
    )(ct)

    BTC = 384
    xt = jnp.transpose(x[BSC:])
    out_tc = pl.pallas_call(
        _onehot_t_body,
        grid=((B - BSC) // BTC,),
        in_specs=[
            pl.BlockSpec((IN, BTC), lambda i: (0, i)),
            pl.BlockSpec((ON, IN * 2 * KN), lambda i: (0, 0)),
            pl.BlockSpec((1, ON), lambda i: (0, 0)),
        ],
        out_specs=pl.BlockSpec((BTC, ON), lambda i: (i, 0)),
        out_shape=jax.ShapeDtypeStruct((B - BSC, ON), jnp.float32),
    )(xt, gt, bias.reshape(1, ON))
    return jnp.concatenate([out_sc, out_tc], axis=0)


def kernel(x, coeffs, bias):
    return _run(x, coeffs, bias)


# speedup vs baseline: 1.5684x; 1.5684x over previous
import functools
import jax
import jax.numpy as jnp
from jax import lax
from jax.experimental import pallas as pl
from jax.experimental.pallas import tpu as pltpu
from jax.experimental.pallas import tpu_sc as plsc

X_MIN = -3.0
X_MAX = 3.0
KN = 64
IN = 64
ON = 256
H = (X_MAX - X_MIN) / (KN - 1)

NW = 32
BSC = 256


def _pchip(y):
    delta = (y[1:, :] - y[:-1, :]) * (1.0 / H)
    d0 = (3.0 * delta[0:1, :] - delta[1:2, :]) * 0.5
    dN = (3.0 * delta[KN - 2:KN - 1, :] - delta[KN - 3:KN - 2, :]) * 0.5

    def fix_end(d_end, delta0, delta1):
        d_end = jnp.where(d_end * delta0 <= 0.0, 0.0, d_end)
        bad = (delta0 * delta1 < 0.0) & (jnp.abs(d_end) > 3.0 * jnp.abs(delta0))
        return jnp.where(bad, 3.0 * delta0, d_end)

    d0 = fix_end(d0, delta[0:1, :], delta[1:2, :])
    dN = fix_end(dN, delta[KN - 2:KN - 1, :], delta[KN - 3:KN - 2, :])
    dp = delta[:-1, :]
    dn = delta[1:, :]
    same = dp * dn > 0.0
    dmid = jnp.where(same, 2.0 * dp * dn / (dp + dn + 1e-12), 0.0)
    return H * jnp.concatenate([d0, dmid, dN], axis=0)


def _prep_body(ct_ref, t1_ref, gt_ref):
    for i in range(IN):
        y = ct_ref[i * KN:(i + 1) * KN, :]
        hd = _pchip(y)
        t1_ref[i * KN:(i + 1) * KN, 0:ON] = y
        t1_ref[i * KN:(i + 1) * KN, ON:2 * ON] = hd
        gt_ref[:, i * 2 * KN:i * 2 * KN + KN] = jnp.transpose(y)
        gt_ref[:, i * 2 * KN + KN:(i + 1) * 2 * KN] = jnp.transpose(hd)


def _weights_body(x_ref, ridx_ref, w_ref):
    x = x_ref[...]
    t = (x - X_MIN) * (1.0 / H)
    idx = jnp.clip(jnp.floor(t), 0.0, float(KN - 2))
    u = t - idx
    u2 = u * u
    u3 = u2 * u
    h00 = 2.0 * u3 - 3.0 * u2 + 1.0
    h10 = u3 - 2.0 * u2 + u
    h01 = 3.0 * u2 - 2.0 * u3
    h11 = u3 - u2
    left = t < 0.0
    right = t > float(KN - 1)
    wy0 = jnp.where(left, 1.0, jnp.where(right, 0.0, h00))
    wd0 = jnp.where(left, t, jnp.where(right, 0.0, h10))
    wy1 = jnp.where(left, 0.0, jnp.where(right, 1.0, h01))
    wd1 = jnp.where(left, 0.0, jnp.where(right, u - 1.0, h11))
    feat = jax.lax.broadcasted_iota(jnp.int32, x.shape, 1)
    rid = feat * KN + idx.astype(jnp.int32)
    ridx_ref[:, 0:IN] = rid
    ridx_ref[:, IN:2 * IN] = rid + 1
    w_ref[:, 0:IN] = wy0
    w_ref[:, IN:2 * IN] = wd0
    w_ref[:, 2 * IN:3 * IN] = wy1
    w_ref[:, 3 * IN:4 * IN] = wd1


def _sc_bag_body(t1_hbm, ridx_hbm, w_hbm, bias_hbm, out_hbm,
                 idx_v, w_v, rows_v, bias_v, out_v, sem_g):
    wid = lax.axis_index("s") * 2 + lax.axis_index("c")
    nb = BSC // NW
    base = wid * nb
    pltpu.sync_copy(bias_hbm, bias_v)

    def body_b(bb, _):
        b = base + bb
        pltpu.sync_copy(ridx_hbm.at[b], idx_v)
        pltpu.sync_copy(w_hbm.at[b], w_v.at[pl.ds(0, 4 * IN)])
        cp0 = pltpu.async_copy(
            t1_hbm.at[idx_v.at[pl.ds(0, IN)]],
            rows_v.at[pl.ds(0, IN)], sem_g)
        cp1 = pltpu.async_copy(
            t1_hbm.at[idx_v.at[pl.ds(IN, IN)]],
            rows_v.at[pl.ds(IN, IN)], sem_g)

        acc0 = tuple(bias_v[pl.ds(c * 16, 16)] for c in range(16))

        def make_body(row_off, wy_off, wd_off):
            def body_i(i, acc):
                wy = jnp.full((16,), w_v[pl.ds(wy_off + i, 16)][0],
                              jnp.float32)
                wd = jnp.full((16,), w_v[pl.ds(wd_off + i, 16)][0],
                              jnp.float32)
                new = []
                for c in range(16):
                    a = acc[c]
                    a = a + wy * rows_v[row_off + i, pl.ds(c * 16, 16)]
                    a = a + wd * rows_v[row_off + i, pl.ds(ON + c * 16, 16)]
                    new.append(a)
                return tuple(new)
            return body_i

        cp0.wait()
        acc = lax.fori_loop(0, IN, make_body(0, 0, IN), acc0)
        cp1.wait()
        acc = lax.fori_loop(0, IN, make_body(IN, 2 * IN, 3 * IN), acc)
        for c in range(16):
            out_v[pl.ds(c * 16, 16)] = acc[c]
        pltpu.sync_copy(out_v, out_hbm.at[b])
        return 0

    lax.fori_loop(0, nb, body_b, 0)


def _onehot_t_body(xt_ref, gt_ref, b_ref, ot_ref):
    x = xt_ref[...]
    t = (x - X_MIN) * (1.0 / H)
    idx = jnp.clip(jnp.floor(t), 0.0, float(KN - 2))
    u = t - idx
    u2 = u * u
    u3 = u2 * u
    h00 = 2.0 * u3 - 3.0 * u2 + 1.0
    h10 = u3 - 2.0 * u2 + u
    h01 = 3.0 * u2 - 2.0 * u3
    h11 = u3 - u2
    left = t < 0.0
    right = t > float(KN - 1)
    wy0 = jnp.where(left, 1.0, jnp.where(right, 0.0, h00))
    wd0 = jnp.where(left, t, jnp.where(right, 0.0, h10))
    wy1 = jnp.where(left, 0.0, jnp.where(right, 1.0, h01))
    wd1 = jnp.where(left, 0.0, jnp.where(right, u - 1.0, h11))

    btc = x.shape[1]
    idx3 = idx.astype(jnp.int32)[:, None, :]
    kk = jax.lax.broadcasted_iota(jnp.int32, (1, 2 * KN, 1), 1)
    kmod = jnp.where(kk < KN, kk, kk - KN)
    isy = kk < KN
    wlo = jnp.where(isy, wy0[:, None, :], wd0[:, None, :])
    whi = jnp.where(isy, wy1[:, None, :], wd1[:, None, :])
    s = jnp.where(kmod == idx3, wlo,
                  jnp.where(kmod == idx3 + 1, whi, 0.0))
    s2 = s.reshape(IN * 2 * KN, btc)
    acc = jax.lax.dot_general(
        gt_ref[...], s2, (((1,), (0,)), ((), ())),
        preferred_element_type=jnp.float32,
        precision=jax.lax.Precision.DEFAULT)
    ot_ref[...] = jnp.transpose(acc) + b_ref[...]


@jax.jit
def _run(x, coeffs, bias):
    ct = jnp.transpose(coeffs, (1, 2, 0)).reshape(IN * KN, ON)
    B = x.shape[0]
    BT = min(512, BSC)
    ridx, w4 = pl.pallas_call(
        _weights_body,
        grid=(BSC // BT,),
        in_specs=[pl.BlockSpec((BT, IN), lambda i: (i, 0))],
        out_specs=[pl.BlockSpec((BT, 2 * IN), lambda i: (i, 0)),
                   pl.BlockSpec((BT, 4 * IN), lambda i: (i, 0))],
        out_shape=[jax.ShapeDtypeStruct((BSC, 2 * IN), jnp.int32),
                   jax.ShapeDtypeStruct((BSC, 4 * IN), jnp.float32)],
    )(x[:BSC])

    t1, gt = pl.pallas_call(
        _prep_body,
        out_shape=[jax.ShapeDtypeStruct((IN * KN, 2 * ON), jnp.float32),
                   jax.ShapeDtypeStruct((ON, IN * 2 * KN), jnp.float32)],
    )(ct)

    mesh = plsc.VectorSubcoreMesh(core_axis_name="c", subcore_axis_name="s")
    sc_bag = functools.partial(
        pl.kernel,
        out_type=jax.ShapeDtypeStruct((BSC, ON), jnp.float32),
        mesh=mesh,
        scratch_types=[
            pltpu.VMEM((2 * IN,), jnp.int32),
            pltpu.VMEM((4 * IN + 16,), jnp.float32),
            pltpu.VMEM((2 * IN, 2 * ON), jnp.float32),
            pltpu.VMEM((ON,), jnp.float32),
            pltpu.VMEM((ON,), jnp.float32),
            pltpu.SemaphoreType.DMA,
        ],
    )(_sc_bag_body)
    out_sc = sc_bag(t1, ridx, w4, bias)

    BTC = 384
    xt = jnp.transpose(x[BSC:])
    out_tc = pl.pallas_call(
        _onehot_t_body,
        grid=((B - BSC) // BTC,),
        in_specs=[
            pl.BlockSpec((IN, BTC), lambda i: (0, i)),
            pl.BlockSpec((ON, IN * 2 * KN), lambda i: (0, 0)),
            pl.BlockSpec((1, ON), lambda i: (0, 0)),
        ],
        out_specs=pl.BlockSpec((BTC, ON), lambda i: (i, 0)),
        out_shape=jax.ShapeDtypeStruct((B - BSC, ON), jnp.float32),
    )(xt, gt, bias.reshape(1, ON))
    return jnp.concatenate([out_sc, out_tc], axis=0)


def kernel(x, coeffs, bias):
    return _run(x, coeffs, bias)


# speedup vs baseline: 1.6668x; 1.0627x over previous
import functools
import jax
import jax.numpy as jnp
from jax import lax
from jax.experimental import pallas as pl
from jax.experimental.pallas import tpu as pltpu
from jax.experimental.pallas import tpu_sc as plsc

X_MIN = -3.0
X_MAX = 3.0
KN = 64
IN = 64
ON = 256
H = (X_MAX - X_MIN) / (KN - 1)

NW = 32
WPAD = 384
BSC = 256


def _pchip(y):
    delta = (y[1:, :] - y[:-1, :]) * (1.0 / H)
    d0 = (3.0 * delta[0:1, :] - delta[1:2, :]) * 0.5
    dN = (3.0 * delta[KN - 2:KN - 1, :] - delta[KN - 3:KN - 2, :]) * 0.5

    def fix_end(d_end, delta0, delta1):
        d_end = jnp.where(d_end * delta0 <= 0.0, 0.0, d_end)
        bad = (delta0 * delta1 < 0.0) & (jnp.abs(d_end) > 3.0 * jnp.abs(delta0))
        return jnp.where(bad, 3.0 * delta0, d_end)

    d0 = fix_end(d0, delta[0:1, :], delta[1:2, :])
    dN = fix_end(dN, delta[KN - 2:KN - 1, :], delta[KN - 3:KN - 2, :])
    dp = delta[:-1, :]
    dn = delta[1:, :]
    same = dp * dn > 0.0
    dmid = jnp.where(same, 2.0 * dp * dn / (dp + dn + 1e-12), 0.0)
    return H * jnp.concatenate([d0, dmid, dN], axis=0)


def _prep_body(ct_ref, t1_ref, gt_ref):
    for i in range(IN):
        y = ct_ref[i * KN:(i + 1) * KN, :]
        hd = _pchip(y)
        t1_ref[i * KN:(i + 1) * KN, 0:ON] = y
        t1_ref[i * KN:(i + 1) * KN, ON:2 * ON] = hd
        gt_ref[:, i * 2 * KN:i * 2 * KN + KN] = jnp.transpose(y)
        gt_ref[:, i * 2 * KN + KN:(i + 1) * 2 * KN] = jnp.transpose(hd)


def _weights_body(x_ref, ridx_ref, w_ref):
    x = x_ref[...]
    t = (x - X_MIN) * (1.0 / H)
    idx = jnp.clip(jnp.floor(t), 0.0, float(KN - 2))
    u = t - idx
    u2 = u * u
    u3 = u2 * u
    h00 = 2.0 * u3 - 3.0 * u2 + 1.0
    h10 = u3 - 2.0 * u2 + u
    h01 = 3.0 * u2 - 2.0 * u3
    h11 = u3 - u2
    left = t < 0.0
    right = t > float(KN - 1)
    wy0 = jnp.where(left, 1.0, jnp.where(right, 0.0, h00))
    wd0 = jnp.where(left, t, jnp.where(right, 0.0, h10))
    wy1 = jnp.where(left, 0.0, jnp.where(right, 1.0, h01))
    wd1 = jnp.where(left, 0.0, jnp.where(right, u - 1.0, h11))
    feat = jax.lax.broadcasted_iota(jnp.int32, x.shape, 1)
    rid = feat * KN + idx.astype(jnp.int32)
    ridx_ref[:, 0:IN] = rid
    ridx_ref[:, IN:2 * IN] = rid + 1
    w_ref[:, 0:IN] = wy0
    w_ref[:, IN:2 * IN] = wd0
    w_ref[:, 2 * IN:3 * IN] = wy1
    w_ref[:, 3 * IN:4 * IN] = wd1
    w_ref[:, 4 * IN:] = jnp.zeros((x.shape[0], WPAD - 4 * IN), x.dtype)


def _sc_bag_body(t1_hbm, ridx_hbm, w_hbm, bias_hbm, out_hbm,
                 idx_v, w_v, rows_v, bias_v, out_v, sem_g):
    wid = lax.axis_index("s") * 2 + lax.axis_index("c")
    nb = BSC // NW
    base = wid * nb
    pltpu.sync_copy(bias_hbm, bias_v)
    pltpu.sync_copy(ridx_hbm.at[pl.ds(base, nb)], idx_v)
    pltpu.sync_copy(w_hbm.at[pl.ds(base * WPAD, nb * WPAD)], w_v)

    def body_b(bb, _):
        b = base + bb
        cp0 = pltpu.async_copy(
            t1_hbm.at[idx_v.at[bb, pl.ds(0, IN)]],
            rows_v.at[pl.ds(0, IN)], sem_g)
        cp1 = pltpu.async_copy(
            t1_hbm.at[idx_v.at[bb, pl.ds(IN, IN)]],
            rows_v.at[pl.ds(IN, IN)], sem_g)

        acc0 = tuple(bias_v[pl.ds(c * 16, 16)] for c in range(16))

        def make_body(row_off, wy_off, wd_off):
            def body_i(i, acc):
                wb = bb * WPAD
                wy = jnp.full((16,), w_v[pl.ds(wb + wy_off + i, 16)][0],
                              jnp.float32)
                wd = jnp.full((16,), w_v[pl.ds(wb + wd_off + i, 16)][0],
                              jnp.float32)
                new = []
                for c in range(16):
                    a = acc[c]
                    a = a + wy * rows_v[row_off + i, pl.ds(c * 16, 16)]
                    a = a + wd * rows_v[row_off + i, pl.ds(ON + c * 16, 16)]
                    new.append(a)
                return tuple(new)
            return body_i

        cp0.wait()
        acc = lax.fori_loop(0, IN, make_body(0, 0, IN), acc0)
        cp1.wait()
        acc = lax.fori_loop(0, IN, make_body(IN, 2 * IN, 3 * IN), acc)
        for c in range(16):
            out_v[pl.ds(c * 16, 16)] = acc[c]
        pltpu.sync_copy(out_v, out_hbm.at[b])
        return 0

    lax.fori_loop(0, nb, body_b, 0)


def _onehot_t_body(xt_ref, gt_ref, b_ref, ot_ref):
    x = xt_ref[...]
    t = (x - X_MIN) * (1.0 / H)
    idx = jnp.clip(jnp.floor(t), 0.0, float(KN - 2))
    u = t - idx
    u2 = u * u
    u3 = u2 * u
    h00 = 2.0 * u3 - 3.0 * u2 + 1.0
    h10 = u3 - 2.0 * u2 + u
    h01 = 3.0 * u2 - 2.0 * u3
    h11 = u3 - u2
    left = t < 0.0
    right = t > float(KN - 1)
    wy0 = jnp.where(left, 1.0, jnp.where(right, 0.0, h00))
    wd0 = jnp.where(left, t, jnp.where(right, 0.0, h10))
    wy1 = jnp.where(left, 0.0, jnp.where(right, 1.0, h01))
    wd1 = jnp.where(left, 0.0, jnp.where(right, u - 1.0, h11))

    btc = x.shape[1]
    idx3 = idx.astype(jnp.int32)[:, None, :]
    kk = jax.lax.broadcasted_iota(jnp.int32, (1, 2 * KN, 1), 1)
    kmod = jnp.where(kk < KN, kk, kk - KN)
    isy = kk < KN
    wlo = jnp.where(isy, wy0[:, None, :], wd0[:, None, :])
    whi = jnp.where(isy, wy1[:, None, :], wd1[:, None, :])
    s = jnp.where(kmod == idx3, wlo,
                  jnp.where(kmod == idx3 + 1, whi, 0.0))
    s2 = s.reshape(IN * 2 * KN, btc)
    acc = jax.lax.dot_general(
        gt_ref[...], s2, (((1,), (0,)), ((), ())),
        preferred_element_type=jnp.float32,
        precision=jax.lax.Precision.DEFAULT)
    ot_ref[...] = jnp.transpose(acc) + b_ref[...]


@jax.jit
def _run(x, coeffs, bias):
    ct = jnp.transpose(coeffs, (1, 2, 0)).reshape(IN * KN, ON)
    B = x.shape[0]
    BT = min(512, BSC)
    ridx, w4 = pl.pallas_call(
        _weights_body,
        grid=(BSC // BT,),
        in_specs=[pl.BlockSpec((BT, IN), lambda i: (i, 0))],
        out_specs=[pl.BlockSpec((BT, 2 * IN), lambda i: (i, 0)),
                   pl.BlockSpec((BT, WPAD), lambda i: (i, 0))],
        out_shape=[jax.ShapeDtypeStruct((BSC, 2 * IN), jnp.int32),
                   jax.ShapeDtypeStruct((BSC, WPAD), jnp.float32)],
    )(x[:BSC])

    t1, gt = pl.pallas_call(
        _prep_body,
        out_shape=[jax.ShapeDtypeStruct((IN * KN, 2 * ON), jnp.float32),
                   jax.ShapeDtypeStruct((ON, IN * 2 * KN), jnp.float32)],
    )(ct)

    mesh = plsc.VectorSubcoreMesh(core_axis_name="c", subcore_axis_name="s")
    sc_bag = functools.partial(
        pl.kernel,
        out_type=jax.ShapeDtypeStruct((BSC, ON), jnp.float32),
        mesh=mesh,
        scratch_types=[
            pltpu.VMEM((BSC // NW, 2 * IN), jnp.int32),
            pltpu.VMEM((BSC // NW * WPAD,), jnp.float32),
            pltpu.VMEM((2 * IN, 2 * ON), jnp.float32),
            pltpu.VMEM((ON,), jnp.float32),
            pltpu.VMEM((ON,), jnp.float32),
            pltpu.SemaphoreType.DMA,
        ],
    )(_sc_bag_body)
    out_sc = sc_bag(t1, ridx, w4.reshape(-1), bias)

    BTC = 384
    xt = jnp.transpose(x[BSC:])
    out_tc = pl.pallas_call(
        _onehot_t_body,
        grid=((B - BSC) // BTC,),
        in_specs=[
            pl.BlockSpec((IN, BTC), lambda i: (0, i)),
            pl.BlockSpec((ON, IN * 2 * KN), lambda i: (0, 0)),
            pl.BlockSpec((1, ON), lambda i: (0, 0)),
        ],
        out_specs=pl.BlockSpec((BTC, ON), lambda i: (i, 0)),
        out_shape=jax.ShapeDtypeStruct((B - BSC, ON), jnp.float32),
    )(xt, gt, bias.reshape(1, ON))
    return jnp.concatenate([out_sc, out_tc], axis=0)


def kernel(x, coeffs, bias):
    return _run(x, coeffs, bias)
